# baseline (speedup 1.0000x reference)
import functools

import jax
import jax.numpy as jnp
from jax import lax
from jax.experimental import pallas as pl
from jax.experimental.pallas import tpu as pltpu
from jax.experimental.pallas import tpu_sc as plsc

NUM_CORES = 2
NUM_SUBCORES = 16
NW = NUM_CORES * NUM_SUBCORES
LANES = 16
CHUNK = 256


def kernel(pos_h, pos_r, pos_t, entity_embds, rel_embds):
    B = pos_h.shape[0]
    E, D = entity_embds.shape
    R = rel_embds.shape[0]
    b_per_w = B // NW

    ent3 = entity_embds.reshape(E // 8, 8, D)
    idx_h = pos_h.astype(jnp.int32)
    idx_r = pos_r.astype(jnp.int32)
    idx_t = pos_t.astype(jnp.int32)

    mesh = plsc.VectorSubcoreMesh(
        core_axis_name="c", subcore_axis_name="s",
        num_cores=NUM_CORES, num_subcores=NUM_SUBCORES,
    )

    out = jax.ShapeDtypeStruct((D // 8, 8, B), jnp.float32)

    @functools.partial(
        pl.kernel,
        out_type=(out, out, out),
        mesh=mesh,
        compiler_params=pltpu.CompilerParams(
            use_tc_tiling_on_sc=True, needs_layout_passes=False),
        scratch_types=[
            pltpu.VMEM((b_per_w,), jnp.int32),
            pltpu.VMEM((b_per_w,), jnp.int32),
            pltpu.VMEM((b_per_w,), jnp.int32),
            pltpu.VMEM((CHUNK, 32), jnp.float32),
            pltpu.VMEM((CHUNK, 32), jnp.float32),
            pltpu.VMEM((CHUNK, 32), jnp.float32),
            pltpu.VMEM((4, 8, CHUNK), jnp.float32),
            pltpu.SemaphoreType.DMA,
            pltpu.SemaphoreType.DMA,
            pltpu.SemaphoreType.DMA,
            pltpu.SemaphoreType.DMA,
        ],
    )
    def run(ih_hbm, ir_hbm, it_hbm, ent_hbm, rel_hbm,
            oh_hbm, or_hbm, ot_hbm,
            ih_v, ir_v, it_v, rows_h, rows_r, rows_t, tbuf,
            sem_h, sem_r, sem_t, sem_out):
        wid = lax.axis_index("s") * NUM_CORES + lax.axis_index("c")
        base = wid * b_per_w

        pltpu.sync_copy(ih_hbm.at[pl.ds(base, b_per_w)], ih_v)
        pltpu.sync_copy(ir_hbm.at[pl.ds(base, b_per_w)], ir_v)
        pltpu.sync_copy(it_hbm.at[pl.ds(base, b_per_w)], it_v)

        def fire_ent(idx_v, rows_v, sem, c):
            def body(g, _):
                vec = idx_v[pl.ds(c * CHUNK + g * LANES, LANES)]
                for l in range(LANES):
                    lane = lax.iota(jnp.int32, LANES) == l
                    row = lax.reduce_sum(jnp.where(lane, vec, 0), axes=(0,))
                    pltpu.async_copy(
                        ent_hbm.at[row >> 3, row & 7],
                        rows_v.at[g * LANES + l], sem)
                return 0

            lax.fori_loop(0, CHUNK // LANES, body, 0)

        def fire_rel(idx_v, rows_v, sem, c):
            def body(g, _):
                vec = idx_v[pl.ds(c * CHUNK + g * LANES, LANES)]
                for l in range(LANES):
                    lane = lax.iota(jnp.int32, LANES) == l
                    row = lax.reduce_sum(jnp.where(lane, vec, 0), axes=(0,))
                    pltpu.async_copy(
                        rel_hbm.at[row], rows_v.at[g * LANES + l], sem)
                return 0

            lax.fori_loop(0, CHUNK // LANES, body, 0)

        def drain(rows_v, sem):
            pltpu.make_async_copy(
                ent_hbm.at[pl.ds(0, CHUNK // 8)], rows_v, sem).wait()

        def flush(rows_v, out3, c):
            def jbody(j, _):
                jsplat = jnp.full((LANES,), 0, jnp.int32) + j
                gsplat = jsplat >> 3
                ssplat = jsplat & 7

                def wbody(w, _):
                    nvec = lax.iota(jnp.int32, LANES) + w * LANES
                    vals = plsc.load_gather(rows_v, [nvec, jsplat])
                    plsc.store_scatter(tbuf, [gsplat, ssplat, nvec], vals)
                    return 0

                lax.fori_loop(0, CHUNK // LANES, wbody, 0)
                return 0

            lax.fori_loop(0, D, jbody, 0)
            off = pl.multiple_of(base + c * CHUNK, 128)
            pltpu.sync_copy(tbuf, out3.at[:, :, pl.ds(off, CHUNK)])

        for c in range(b_per_w // CHUNK):
            fire_ent(ih_v, rows_h, sem_h, c)
            fire_ent(it_v, rows_t, sem_t, c)
            fire_rel(ir_v, rows_r, sem_r, c)
            drain(rows_h, sem_h)
            flush(rows_h, oh_hbm, c)
            drain(rows_t, sem_t)
            flush(rows_t, ot_hbm, c)
            drain(rows_r, sem_r)
            flush(rows_r, or_hbm, c)

    oh3, or3, ot3 = run(idx_h, idx_r, idx_t, ent3, rel_embds)
    return (oh3.reshape(D, B).T, or3.reshape(D, B).T, ot3.reshape(D, B).T)


# speedup vs baseline: 1.0021x; 1.0021x over previous
import functools

import jax
import jax.numpy as jnp
from jax import lax
from jax.experimental import pallas as pl
from jax.experimental.pallas import tpu as pltpu
from jax.experimental.pallas import tpu_sc as plsc

NUM_CORES = 2
NUM_SUBCORES = 16
NW = NUM_CORES * NUM_SUBCORES
LANES = 16
CHUNK = 256


def kernel(pos_h, pos_r, pos_t, entity_embds, rel_embds):
    B = pos_h.shape[0]
    E, D = entity_embds.shape
    R = rel_embds.shape[0]
    b_per_w = B // NW

    ent3 = entity_embds.reshape(E // 8, 8, D)
    idx_h = pos_h.astype(jnp.int32)
    idx_r = pos_r.astype(jnp.int32)
    idx_t = pos_t.astype(jnp.int32)

    mesh = plsc.VectorSubcoreMesh(
        core_axis_name="c", subcore_axis_name="s",
        num_cores=NUM_CORES, num_subcores=NUM_SUBCORES,
    )

    out = jax.ShapeDtypeStruct((D // 8, 8, B), jnp.float32)

    @functools.partial(
        pl.kernel,
        out_type=(out, out, out),
        mesh=mesh,
        compiler_params=pltpu.CompilerParams(
            use_tc_tiling_on_sc=True, needs_layout_passes=False),
        scratch_types=[
            pltpu.VMEM((b_per_w,), jnp.int32),
            pltpu.VMEM((b_per_w,), jnp.int32),
            pltpu.VMEM((b_per_w,), jnp.int32),
            pltpu.VMEM((CHUNK, D), jnp.float32),
            pltpu.VMEM((CHUNK, D), jnp.float32),
            pltpu.VMEM((CHUNK, D), jnp.float32),
            pltpu.VMEM((D // 8, 8, CHUNK), jnp.float32),
            pltpu.SemaphoreType.DMA,
            pltpu.SemaphoreType.DMA,
            pltpu.SemaphoreType.DMA,
            pltpu.SemaphoreType.DMA,
        ],
    )
    def run(ih_hbm, ir_hbm, it_hbm, ent_hbm, rel_hbm,
            oh_hbm, or_hbm, ot_hbm,
            ih_v, ir_v, it_v, rows_h, rows_r, rows_t, tbuf,
            sem_h, sem_r, sem_t, sem_out):
        wid = lax.axis_index("s") * NUM_CORES + lax.axis_index("c")
        base = wid * b_per_w

        pltpu.sync_copy(ih_hbm.at[pl.ds(base, b_per_w)], ih_v)
        pltpu.sync_copy(ir_hbm.at[pl.ds(base, b_per_w)], ir_v)
        pltpu.sync_copy(it_hbm.at[pl.ds(base, b_per_w)], it_v)

        def fire_ent(idx_v, rows_v, sem, c):
            def body(g, _):
                vec = idx_v[pl.ds(c * CHUNK + g * LANES, LANES)]
                for l in range(LANES):
                    lane = lax.iota(jnp.int32, LANES) == l
                    row = lax.reduce_sum(jnp.where(lane, vec, 0), axes=(0,))
                    pltpu.async_copy(
                        ent_hbm.at[row >> 3, row & 7],
                        rows_v.at[g * LANES + l], sem)
                return 0

            lax.fori_loop(0, CHUNK // LANES, body, 0)

        def fire_rel(idx_v, rows_v, sem, c):
            def body(g, _):
                vec = idx_v[pl.ds(c * CHUNK + g * LANES, LANES)]
                for l in range(LANES):
                    lane = lax.iota(jnp.int32, LANES) == l
                    row = lax.reduce_sum(jnp.where(lane, vec, 0), axes=(0,))
                    pltpu.async_copy(
                        rel_hbm.at[row], rows_v.at[g * LANES + l], sem)
                return 0

            lax.fori_loop(0, CHUNK // LANES, body, 0)

        def drain(rows_v, sem):
            pltpu.make_async_copy(
                ent_hbm.at[pl.ds(0, CHUNK // 8)], rows_v, sem).wait()

        def flush(rows_v, out3, c):
            def jbody(j, _):
                jsplat = jnp.full((LANES,), 0, jnp.int32) + j
                gsplat = jsplat >> 3
                ssplat = jsplat & 7

                def wbody(w, _):
                    nvec = lax.iota(jnp.int32, LANES) + w * LANES
                    vals = plsc.load_gather(rows_v, [nvec, jsplat])
                    plsc.store_scatter(tbuf, [gsplat, ssplat, nvec], vals)
                    return 0

                lax.fori_loop(0, CHUNK // LANES, wbody, 0)
                return 0

            lax.fori_loop(0, D, jbody, 0)
            off = pl.multiple_of(base + c * CHUNK, 128)
            pltpu.sync_copy(tbuf, out3.at[:, :, pl.ds(off, CHUNK)])

        for c in range(b_per_w // CHUNK):
            fire_ent(ih_v, rows_h, sem_h, c)
            fire_ent(it_v, rows_t, sem_t, c)
            fire_rel(ir_v, rows_r, sem_r, c)
            drain(rows_h, sem_h)
            flush(rows_h, oh_hbm, c)
            drain(rows_t, sem_t)
            flush(rows_t, ot_hbm, c)
            drain(rows_r, sem_r)
            flush(rows_r, or_hbm, c)

    oh3, or3, ot3 = run(idx_h, idx_r, idx_t, ent3, rel_embds)
    return (oh3.reshape(D, B).T, or3.reshape(D, B).T, ot3.reshape(D, B).T)
